# baseline (device time: 15107 ns/iter reference)
import jax
import jax.numpy as jnp
from jax import lax
from jax.experimental import pallas as pl
from jax.experimental.pallas import tpu as pltpu

P = 4


def kernel(Q, K, V):
    b, sq, h, d = Q.shape
    skv = K.shape[1]
    scale = d ** -0.5

    Kt = jnp.transpose(K, (0, 2, 3, 1))
    Vt = jnp.transpose(V, (0, 2, 3, 1))

    qk = skv // 4
    x_idx = lax.axis_index("x")
    z_par = lax.axis_index("z") % 2
    start = x_idx * (skv // 2) + z_par * qk
    Kt = lax.dynamic_slice_in_dim(Kt, start, qk, axis=3)
    Vt = lax.dynamic_slice_in_dim(Vt, start, qk, axis=3)

    def body(q_ref, kt_ref, vt_ref, o_ref,
             zsend_buf, zrecv_buf, send_buf, recv_bufs,
             zsend_sem, zrecv_sem, local_sem, send_sems, recv_sems):
        my_x = lax.axis_index("x")
        my_y = lax.axis_index("y")
        my_z = lax.axis_index("z")
        pid = my_x * 2 + my_y
        zn = (my_z // 2) * 2 + (1 - my_z % 2)

        barrier_sem = pltpu.get_barrier_semaphore()
        for off in range(1, P):
            pfid = (pid + off) % P
            pl.semaphore_signal(
                barrier_sem, inc=1,
                device_id=(pfid // 2, pfid % 2, my_z),
                device_id_type=pl.DeviceIdType.MESH,
            )
        pl.semaphore_signal(
            barrier_sem, inc=1, device_id=(my_x, my_y, zn),
            device_id_type=pl.DeviceIdType.MESH,
        )
        pl.semaphore_wait(barrier_sem, P)

        q = q_ref[:, 0, :, :]
        kt = kt_ref[...]
        vt = vt_ref[...]

        s = jnp.sum(q[..., None] * kt, axis=2) * scale
        p = jnp.exp(s)
        l_c = jnp.sum(p, axis=-1)
        o_c = jnp.sum(p[:, :, None, :] * vt, axis=-1)

        zsend_buf[0, :, :, :] = o_c
        zsend_buf[1, :, :, :] = jnp.broadcast_to(l_c[:, :, None], (b, h, d))
        zrd = pltpu.make_async_remote_copy(
            src_ref=zsend_buf,
            dst_ref=zrecv_buf,
            send_sem=zsend_sem,
            recv_sem=zrecv_sem,
            device_id=(my_x, my_y, zn),
            device_id_type=pl.DeviceIdType.MESH,
        )
        zrd.start()
        zrd.wait()

        send_buf[...] = zsend_buf[...] + zrecv_buf[...]
        sends = []
        for off in range(1, P):
            pfid = (pid + off) % P
            rd = pltpu.make_async_remote_copy(
                src_ref=send_buf,
                dst_ref=recv_bufs.at[pid],
                send_sem=send_sems.at[pfid],
                recv_sem=recv_sems.at[pid],
                device_id=(pfid // 2, pfid % 2, my_z),
                device_id_type=pl.DeviceIdType.MESH,
            )
            rd.start()
            sends.append(rd)
        cp_self = pltpu.make_async_copy(send_buf, recv_bufs.at[pid], local_sem)
        cp_self.start()

        for off in range(1, P):
            pfid = (pid + off) % P
            pltpu.make_async_remote_copy(
                src_ref=send_buf,
                dst_ref=recv_bufs.at[pfid],
                send_sem=send_sems.at[pfid],
                recv_sem=recv_sems.at[pfid],
                device_id=(pfid // 2, pfid % 2, my_z),
                device_id_type=pl.DeviceIdType.MESH,
            ).wait_recv()
        cp_self.wait()

        tot = jnp.sum(recv_bufs[...], axis=0)
        o_ref[:, 0, :, :] = tot[0] / tot[1]

        for rd in sends:
            rd.wait_send()

    return pl.pallas_call(
        body,
        out_shape=jax.ShapeDtypeStruct((b, sq, h, d), jnp.float32),
        in_specs=[
            pl.BlockSpec(memory_space=pltpu.VMEM),
            pl.BlockSpec(memory_space=pltpu.VMEM),
            pl.BlockSpec(memory_space=pltpu.VMEM),
        ],
        out_specs=pl.BlockSpec(memory_space=pltpu.VMEM),
        scratch_shapes=[
            pltpu.VMEM((2, b, h, d), jnp.float32),
            pltpu.VMEM((2, b, h, d), jnp.float32),
            pltpu.VMEM((2, b, h, d), jnp.float32),
            pltpu.VMEM((P, 2, b, h, d), jnp.float32),
            pltpu.SemaphoreType.DMA,
            pltpu.SemaphoreType.DMA,
            pltpu.SemaphoreType.DMA,
            pltpu.SemaphoreType.DMA((P,)),
            pltpu.SemaphoreType.DMA((P,)),
        ],
        compiler_params=pltpu.CompilerParams(collective_id=0),
    )(Q, Kt, Vt)
